# Initial kernel scaffold; baseline (speedup 1.0000x reference)
import functools

import jax
import jax.numpy as jnp
from jax import lax
from jax.experimental import pallas as pl
from jax.experimental.pallas import tpu as pltpu
from jax.experimental.pallas import tpu_sc as plsc

L = 4096
NB = 65
HIST_PAD = 4240
NLANE = 16
NVREG = L // NLANE
NCORES = 2
NSUB = 16
NWORKERS = NCORES * NSUB
RADIX_BITS = (11, 11, 10)
RADIX = 1 << 11

_I32_MIN = jnp.int32(-2147483648)
_I32_MAX = jnp.int32(2147483647)


def _row_kernel(x_hbm, out_hbm, xv, keys0, pay0, keys1, pay1, hist, bins,
                rowhist):
  total_rows = x_hbm.shape[0]
  rows_per_worker = total_rows // NWORKERS
  wid = lax.axis_index("s") * NCORES + lax.axis_index("c")
  iota = lax.iota(jnp.int32, NLANE)
  zeros16i = jnp.zeros((NLANE,), jnp.int32)
  zeros16f = jnp.zeros((NLANE,), jnp.float32)

  bins[pl.ds(L, NLANE)] = zeros16i

  def radix_pass(shift, nbits, kin, pin, kout, pout):
    mask = jnp.int32((1 << nbits) - 1)
    nhist = 1 << nbits

    def clear_body(j, _):
      hist[pl.ds(j * NLANE, NLANE)] = zeros16i
      return 0

    lax.fori_loop(0, nhist // NLANE, clear_body, 0, unroll=4)

    def count_body(j, _):
      k = kin[pl.ds(j * NLANE, NLANE)]
      d = lax.shift_right_logical(k, shift) & mask
      occ, last = plsc.scan_count(d)
      plsc.addupdate_scatter(hist, [d], occ, mask=last)
      return 0

    lax.fori_loop(0, NVREG, count_body, 0, unroll=4)

    def scan_body(j, carry):
      v = hist[pl.ds(j * NLANE, NLANE)]
      inc = plsc.cumsum(v)
      hist[pl.ds(j * NLANE, NLANE)] = inc - v + carry
      return carry + jnp.sum(v)

    lax.fori_loop(0, nhist // NLANE, scan_body, jnp.int32(0), unroll=2)

    def permute_body(j, _):
      k = kin[pl.ds(j * NLANE, NLANE)]
      p = pin[pl.ds(j * NLANE, NLANE)]
      d = lax.shift_right_logical(k, shift) & mask
      occ, last = plsc.scan_count(d)
      base = plsc.load_gather(hist, [d])
      slot = base + occ - 1
      plsc.store_scatter(kout, [slot], k)
      plsc.store_scatter(pout, [slot], p)
      plsc.addupdate_scatter(hist, [d], occ, mask=last)
      return 0

    lax.fori_loop(0, NVREG, permute_body, 0, unroll=2)

  def row_body(r, _):
    row = wid * rows_per_worker + r
    pltpu.sync_copy(x_hbm.at[row], xv)

    def valid_body(j, carry):
      mn, mx = carry
      v = xv[pl.ds(j * NLANE, NLANE)]
      nz = v != 0.0
      idx = j * NLANE + iota
      mn = jnp.minimum(mn, jnp.min(jnp.where(nz, idx, jnp.int32(L))))
      mx = jnp.maximum(mx, jnp.max(jnp.where(nz, idx, jnp.int32(-1))))
      return mn, mx

    start, end = lax.fori_loop(0, NVREG, valid_body,
                               (jnp.int32(L), jnp.int32(-1)), unroll=4)
    vlen = end - start + 1
    lenc = jnp.maximum(vlen, 1)

    def key_body(j, _):
      v = xv[pl.ds(j * NLANE, NLANE)]
      t = plsc.bitcast(v, jnp.int32)
      s = t ^ (lax.shift_right_arithmetic(t, 31) & _I32_MAX)
      u = s ^ _I32_MIN
      idx = j * NLANE + iota
      ok = (idx >= start) & (idx <= end)
      keys0[pl.ds(j * NLANE, NLANE)] = jnp.where(ok, u, jnp.int32(-1))
      pay0[pl.ds(j * NLANE, NLANE)] = idx
      return 0

    lax.fori_loop(0, NVREG, key_body, 0, unroll=4)

    radix_pass(0, RADIX_BITS[0], keys0, pay0, keys1, pay1)
    radix_pass(11, RADIX_BITS[1], keys1, pay1, keys0, pay0)
    radix_pass(22, RADIX_BITS[2], keys0, pay0, keys1, pay1)

    def bin_body(j, _):
      p = pay1[pl.ds(j * NLANE, NLANE)]
      pos = j * NLANE + iota
      b = jnp.minimum((pos * NB) // lenc, NB - 1)
      plsc.store_scatter(bins, [p], b)
      return 0

    lax.fori_loop(0, NVREG, bin_body, 0, unroll=4)

    def hclear_body(j, _):
      rowhist[pl.ds(j * NLANE, NLANE)] = zeros16f
      return 0

    lax.fori_loop(0, HIST_PAD // NLANE, hclear_body, 0, unroll=4)

    def trans_body(j, _):
      a = bins[pl.ds(j * NLANE, NLANE)]
      b = bins[pl.ds(j * NLANE + 1, NLANE)]
      t = j * NLANE + iota
      ok = (t >= start) & (t <= end - 1)
      cell = a * NB + b
      occ, last = plsc.scan_count(cell, mask=ok)
      plsc.addupdate_scatter(rowhist, [cell], occ.astype(jnp.float32),
                             mask=last & ok)
      return 0

    lax.fori_loop(0, NVREG, trans_body, 0, unroll=2)

    inv = 1.0 / jnp.maximum(vlen - 1, 1).astype(jnp.float32)

    def norm_body(j, _):
      sl = pl.ds(j * NLANE, NLANE)
      rowhist[sl] = rowhist[sl] * inv
      return 0

    lax.fori_loop(0, HIST_PAD // NLANE, norm_body, 0, unroll=4)

    pltpu.sync_copy(rowhist, out_hbm.at[row])
    return 0

  lax.fori_loop(0, rows_per_worker, row_body, 0)


@jax.jit
def kernel(x):
  N, C, Lx = x.shape
  rows = N * C
  x2 = x.reshape(rows, Lx)
  mesh = plsc.VectorSubcoreMesh(core_axis_name="c", subcore_axis_name="s",
                                num_cores=NCORES, num_subcores=NSUB)
  run = functools.partial(
      pl.kernel,
      mesh=mesh,
      out_type=jax.ShapeDtypeStruct((rows, HIST_PAD), jnp.float32),
      scratch_types=[
          pltpu.VMEM((L,), jnp.float32),
          pltpu.VMEM((L,), jnp.int32),
          pltpu.VMEM((L,), jnp.int32),
          pltpu.VMEM((L,), jnp.int32),
          pltpu.VMEM((L,), jnp.int32),
          pltpu.VMEM((RADIX,), jnp.int32),
          pltpu.VMEM((L + NLANE,), jnp.int32),
          pltpu.VMEM((HIST_PAD,), jnp.float32),
      ],
  )(_row_kernel)
  out = run(x2)
  return out[:, :NB * NB].reshape(N, C, NB, NB)


# speedup vs baseline: 8.8387x; 8.8387x over previous
import functools

import jax
import jax.numpy as jnp
from jax import lax
from jax.experimental import pallas as pl
from jax.experimental.pallas import tpu as pltpu
from jax.experimental.pallas import tpu_sc as plsc

L = 4096
NB = 65
HIST_PAD = 4240
NLANE = 16
NVREG = L // NLANE
NCORES = 2
NSUB = 16
NWORKERS = NCORES * NSUB
RADIX_BITS = (11, 11, 10)
RADIX = 1 << 11

_I32_MIN = -2147483648
_I32_MAX = 2147483647


def _row_kernel(x_hbm, out_hbm, xv, keys0, pay0, keys1, pay1, hist, bins,
                rowhist):
  total_rows = x_hbm.shape[0]
  rows_per_worker = total_rows // NWORKERS
  wid = lax.axis_index("s") * NCORES + lax.axis_index("c")
  iota = lax.iota(jnp.int32, NLANE)
  zeros16i = jnp.zeros((NLANE,), jnp.int32)
  zeros16f = jnp.zeros((NLANE,), jnp.float32)

  bins[pl.ds(L, NLANE)] = zeros16i

  def radix_pass(shift, nbits, kin, pin, kout, pout):
    mask = jnp.int32((1 << nbits) - 1)
    nhist = 1 << nbits

    def clear_body(j, _):
      hist[pl.ds(j * NLANE, NLANE)] = zeros16i
      return 0

    lax.fori_loop(0, nhist // NLANE, clear_body, 0, unroll=4)

    def count_body(j, _):
      k = kin[pl.ds(j * NLANE, NLANE)]
      d = lax.shift_right_logical(k, shift) & mask
      occ, last = plsc.scan_count(d)
      plsc.addupdate_scatter(hist, [d], occ, mask=last)
      return 0

    lax.fori_loop(0, NVREG, count_body, 0, unroll=4)

    def scan_body(j, carry):
      v = hist[pl.ds(j * NLANE, NLANE)]
      inc = plsc.cumsum(v)
      hist[pl.ds(j * NLANE, NLANE)] = inc - v + carry
      return carry + jnp.sum(v)

    lax.fori_loop(0, nhist // NLANE, scan_body, jnp.int32(0), unroll=2)

    def permute_body(j, _):
      k = kin[pl.ds(j * NLANE, NLANE)]
      p = pin[pl.ds(j * NLANE, NLANE)]
      d = lax.shift_right_logical(k, shift) & mask
      occ, last = plsc.scan_count(d)
      base = plsc.load_gather(hist, [d])
      slot = base + occ - 1
      plsc.store_scatter(kout, [slot], k)
      plsc.store_scatter(pout, [slot], p)
      plsc.addupdate_scatter(hist, [d], occ, mask=last)
      return 0

    lax.fori_loop(0, NVREG, permute_body, 0, unroll=2)

  def row_body(r, _):
    row = wid * rows_per_worker + r
    pltpu.sync_copy(x_hbm.at[row], xv)

    def valid_body(j, carry):
      mn, mx = carry
      v = xv[pl.ds(j * NLANE, NLANE)]
      nz = v != 0.0
      idx = j * NLANE + iota
      mn = jnp.minimum(mn, jnp.min(jnp.where(nz, idx, jnp.int32(L))))
      mx = jnp.maximum(mx, jnp.max(jnp.where(nz, idx, jnp.int32(-1))))
      return mn, mx

    start, end = lax.fori_loop(0, NVREG, valid_body,
                               (jnp.int32(L), jnp.int32(-1)), unroll=4)
    vlen = end - start + 1
    lenc = jnp.maximum(vlen, 1)

    def key_body(j, _):
      v = xv[pl.ds(j * NLANE, NLANE)]
      t = plsc.bitcast(v, jnp.int32)
      s = t ^ (lax.shift_right_arithmetic(t, 31) & _I32_MAX)
      u = s ^ _I32_MIN
      idx = j * NLANE + iota
      ok = (idx >= start) & (idx <= end)
      keys0[pl.ds(j * NLANE, NLANE)] = jnp.where(ok, u, jnp.int32(-1))
      pay0[pl.ds(j * NLANE, NLANE)] = idx
      return 0

    lax.fori_loop(0, NVREG, key_body, 0, unroll=4)

    radix_pass(0, RADIX_BITS[0], keys0, pay0, keys1, pay1)
    radix_pass(11, RADIX_BITS[1], keys1, pay1, keys0, pay0)
    radix_pass(22, RADIX_BITS[2], keys0, pay0, keys1, pay1)

    def bin_body(j, _):
      p = pay1[pl.ds(j * NLANE, NLANE)]
      pos = j * NLANE + iota
      b = jnp.minimum((pos * NB) // lenc, NB - 1)
      plsc.store_scatter(bins, [p], b)
      return 0

    lax.fori_loop(0, NVREG, bin_body, 0, unroll=4)

    def hclear_body(j, _):
      rowhist[pl.ds(j * NLANE, NLANE)] = zeros16f
      return 0

    lax.fori_loop(0, HIST_PAD // NLANE, hclear_body, 0, unroll=4)

    def trans_body(j, _):
      a = bins[pl.ds(j * NLANE, NLANE)]
      b = bins[pl.ds(j * NLANE + 1, NLANE)]
      t = j * NLANE + iota
      ok = (t >= start) & (t <= end - 1)
      cell = a * NB + b
      occ, last = plsc.scan_count(cell, mask=ok)
      plsc.addupdate_scatter(rowhist, [cell], occ.astype(jnp.float32),
                             mask=last & ok)
      return 0

    lax.fori_loop(0, NVREG, trans_body, 0, unroll=2)

    denom = (zeros16i + jnp.maximum(vlen - 1, 1)).astype(jnp.float32)
    inv = (zeros16f + 1.0) / denom

    def norm_body(j, _):
      sl = pl.ds(j * NLANE, NLANE)
      rowhist[sl] = rowhist[sl] * inv
      return 0

    lax.fori_loop(0, HIST_PAD // NLANE, norm_body, 0, unroll=4)

    pltpu.sync_copy(rowhist, out_hbm.at[row])
    return 0

  lax.fori_loop(0, rows_per_worker, row_body, 0)


@jax.jit
def kernel(x):
  N, C, Lx = x.shape
  rows = N * C
  x2 = x.reshape(rows, Lx)
  mesh = plsc.VectorSubcoreMesh(core_axis_name="c", subcore_axis_name="s",
                                num_cores=NCORES, num_subcores=NSUB)
  run = functools.partial(
      pl.kernel,
      mesh=mesh,
      compiler_params=pltpu.CompilerParams(needs_layout_passes=False),
      out_type=jax.ShapeDtypeStruct((rows, HIST_PAD), jnp.float32),
      scratch_types=[
          pltpu.VMEM((L,), jnp.float32),
          pltpu.VMEM((L,), jnp.int32),
          pltpu.VMEM((L,), jnp.int32),
          pltpu.VMEM((L,), jnp.int32),
          pltpu.VMEM((L,), jnp.int32),
          pltpu.VMEM((RADIX,), jnp.int32),
          pltpu.VMEM((L + NLANE,), jnp.int32),
          pltpu.VMEM((HIST_PAD,), jnp.float32),
      ],
  )(_row_kernel)
  out = run(x2)
  return out[:, :NB * NB].reshape(N, C, NB, NB)


# speedup vs baseline: 10.4841x; 1.1862x over previous
import functools

import jax
import jax.numpy as jnp
from jax import lax
from jax.experimental import pallas as pl
from jax.experimental.pallas import tpu as pltpu
from jax.experimental.pallas import tpu_sc as plsc

L = 4096
NB = 65
HIST_PAD = 4240
NLANE = 16
NVREG = L // NLANE
NCORES = 2
NSUB = 16
NWORKERS = NCORES * NSUB
RADIX = 1 << 11

_I32_MIN = -2147483648
_I32_MAX = 2147483647


def _row_kernel(x_hbm, out_hbm, xv, keys0, pay0, keys1, pay1, hist0, hist1,
                bins, rowhist):
  total_rows = x_hbm.shape[0]
  rows_per_worker = total_rows // NWORKERS
  wid = lax.axis_index("s") * NCORES + lax.axis_index("c")
  iota = lax.iota(jnp.int32, NLANE)
  zeros16i = jnp.zeros((NLANE,), jnp.int32)
  zeros16f = jnp.zeros((NLANE,), jnp.float32)

  bins[pl.ds(L, NLANE)] = zeros16i

  def exclusive_scan(src, dst_clear, n, m):
    def body(j, carry):
      v = src[pl.ds(j * NLANE, NLANE)]
      inc = plsc.cumsum(v)
      src[pl.ds(j * NLANE, NLANE)] = inc - v + carry

      @pl.when(j < m // NLANE)
      def _():
        dst_clear[pl.ds(j * NLANE, NLANE)] = zeros16i

      return carry + jnp.sum(v)

    lax.fori_loop(0, n // NLANE, body, jnp.int32(0), unroll=2)

  def row_body(r, _):
    row = wid * rows_per_worker + r
    pltpu.sync_copy(x_hbm.at[row], xv)

    def valid_body(j, carry):
      fv, lv = carry
      v = xv[pl.ds(j * NLANE, NLANE)]
      nz = v != 0.0
      idxv = j * NLANE + iota
      fv = jnp.minimum(fv, jnp.where(nz, idxv, jnp.int32(L)))
      lv = jnp.maximum(lv, jnp.where(nz, idxv, jnp.int32(-1)))

      @pl.when(j < RADIX // NLANE)
      def _():
        hist0[pl.ds(j * NLANE, NLANE)] = zeros16i

      return fv, lv

    fv, lv = lax.fori_loop(0, NVREG, valid_body,
                           (zeros16i + L, zeros16i - 1), unroll=4)
    start = jnp.min(fv)
    end = jnp.max(lv)
    vlen = end - start + 1
    lenc = jnp.maximum(vlen, 1)

    def key_body(j, _):
      v = xv[pl.ds(j * NLANE, NLANE)]
      t = plsc.bitcast(v, jnp.int32)
      s = t ^ (lax.shift_right_arithmetic(t, 31) & _I32_MAX)
      u = s ^ _I32_MIN
      idxv = j * NLANE + iota
      ok = (idxv >= start) & (idxv <= end)
      key = jnp.where(ok, u, jnp.int32(-1))
      keys0[pl.ds(j * NLANE, NLANE)] = key
      pay0[pl.ds(j * NLANE, NLANE)] = idxv
      d = key & (RADIX - 1)
      occ, last = plsc.scan_count(d)
      plsc.addupdate_scatter(hist0, [d], occ, mask=last)
      return 0

    lax.fori_loop(0, NVREG, key_body, 0, unroll=4)

    exclusive_scan(hist0, hist1, RADIX, RADIX)

    def permute0_body(j, _):
      k = keys0[pl.ds(j * NLANE, NLANE)]
      p = pay0[pl.ds(j * NLANE, NLANE)]
      d = k & (RADIX - 1)
      occ, last = plsc.scan_count(d)
      base = plsc.load_gather(hist0, [d])
      slot = base + occ - 1
      plsc.store_scatter(keys1, [slot], k)
      plsc.store_scatter(pay1, [slot], p)
      plsc.addupdate_scatter(hist0, [d], occ, mask=last)
      d1 = lax.shift_right_logical(k, 11) & (RADIX - 1)
      occ1, last1 = plsc.scan_count(d1)
      plsc.addupdate_scatter(hist1, [d1], occ1, mask=last1)
      return 0

    lax.fori_loop(0, NVREG, permute0_body, 0, unroll=2)

    exclusive_scan(hist1, hist0, RADIX, 1024)

    def permute1_body(j, _):
      k = keys1[pl.ds(j * NLANE, NLANE)]
      p = pay1[pl.ds(j * NLANE, NLANE)]
      d = lax.shift_right_logical(k, 11) & (RADIX - 1)
      occ, last = plsc.scan_count(d)
      base = plsc.load_gather(hist1, [d])
      slot = base + occ - 1
      plsc.store_scatter(keys0, [slot], k)
      plsc.store_scatter(pay0, [slot], p)
      plsc.addupdate_scatter(hist1, [d], occ, mask=last)
      d2 = lax.shift_right_logical(k, 22) & 1023
      occ2, last2 = plsc.scan_count(d2)
      plsc.addupdate_scatter(hist0, [d2], occ2, mask=last2)
      return 0

    lax.fori_loop(0, NVREG, permute1_body, 0, unroll=2)

    exclusive_scan(hist0, hist1, 1024, 0)

    def hclear_body(j, _):
      rowhist[pl.ds(j * NLANE, NLANE)] = zeros16f
      return 0

    lax.fori_loop(0, HIST_PAD // NLANE, hclear_body, 0, unroll=4)

    def permute2_body(j, _):
      k = keys0[pl.ds(j * NLANE, NLANE)]
      p = pay0[pl.ds(j * NLANE, NLANE)]
      d = lax.shift_right_logical(k, 22) & 1023
      occ, last = plsc.scan_count(d)
      base = plsc.load_gather(hist0, [d])
      slot = base + occ - 1
      plsc.addupdate_scatter(hist0, [d], occ, mask=last)
      b = jnp.minimum((slot * NB) // lenc, NB - 1)
      plsc.store_scatter(bins, [p], b)
      return 0

    lax.fori_loop(0, NVREG, permute2_body, 0, unroll=2)

    def trans_body(j, _):
      a = bins[pl.ds(j * NLANE, NLANE)]
      b = bins[pl.ds(j * NLANE + 1, NLANE)]
      t = j * NLANE + iota
      ok = (t >= start) & (t <= end - 1)
      cell = a * NB + b
      occ, last = plsc.scan_count(cell, mask=ok)
      plsc.addupdate_scatter(rowhist, [cell], occ.astype(jnp.float32),
                             mask=last & ok)
      return 0

    lax.fori_loop(0, NVREG, trans_body, 0, unroll=4)

    denom = (zeros16i + jnp.maximum(vlen - 1, 1)).astype(jnp.float32)
    inv = (zeros16f + 1.0) / denom

    def norm_body(j, _):
      sl = pl.ds(j * NLANE, NLANE)
      rowhist[sl] = rowhist[sl] * inv
      return 0

    lax.fori_loop(0, HIST_PAD // NLANE, norm_body, 0, unroll=4)

    pltpu.sync_copy(rowhist, out_hbm.at[row])
    return 0

  lax.fori_loop(0, rows_per_worker, row_body, 0)


@jax.jit
def kernel(x):
  N, C, Lx = x.shape
  rows = N * C
  x2 = x.reshape(rows, Lx)
  mesh = plsc.VectorSubcoreMesh(core_axis_name="c", subcore_axis_name="s",
                                num_cores=NCORES, num_subcores=NSUB)
  run = functools.partial(
      pl.kernel,
      mesh=mesh,
      compiler_params=pltpu.CompilerParams(needs_layout_passes=False),
      out_type=jax.ShapeDtypeStruct((rows, HIST_PAD), jnp.float32),
      scratch_types=[
          pltpu.VMEM((L,), jnp.float32),
          pltpu.VMEM((L,), jnp.int32),
          pltpu.VMEM((L,), jnp.int32),
          pltpu.VMEM((L,), jnp.int32),
          pltpu.VMEM((L,), jnp.int32),
          pltpu.VMEM((RADIX,), jnp.int32),
          pltpu.VMEM((RADIX,), jnp.int32),
          pltpu.VMEM((L + NLANE,), jnp.int32),
          pltpu.VMEM((HIST_PAD,), jnp.float32),
      ],
  )(_row_kernel)
  out = run(x2)
  return out[:, :NB * NB].reshape(N, C, NB, NB)


# speedup vs baseline: 14.5012x; 1.3832x over previous
import functools

import jax
import jax.numpy as jnp
from jax import lax
from jax.experimental import pallas as pl
from jax.experimental.pallas import tpu as pltpu
from jax.experimental.pallas import tpu_sc as plsc

L = 4096
NB = 65
HIST_PAD = 4240
NLANE = 16
NVREG = L // NLANE
NCORES = 2
NSUB = 16
NWORKERS = NCORES * NSUB
RADIX = 1 << 11

_I32_MIN = -2147483648
_I32_MAX = 2147483647


def _row_kernel(x_hbm, out_hbm, xv, keys0, pay0, keys1, pay1, hist0, hist1,
                bins, rowhist):
  total_rows = x_hbm.shape[0]
  rows_per_worker = total_rows // NWORKERS
  wid = lax.axis_index("s") * NCORES + lax.axis_index("c")
  iota = lax.iota(jnp.int32, NLANE)
  zeros16i = jnp.zeros((NLANE,), jnp.int32)
  zeros16f = jnp.zeros((NLANE,), jnp.float32)

  bins[pl.ds(L, NLANE)] = zeros16i

  def exclusive_scan(src, dst_clear, n, m):
    def body(j, carry):
      v = src[pl.ds(j * NLANE, NLANE)]
      inc = plsc.cumsum(v)
      src[pl.ds(j * NLANE, NLANE)] = inc - v + carry

      @pl.when(j < m // NLANE)
      def _():
        dst_clear[pl.ds(j * NLANE, NLANE)] = zeros16i

      return carry + jnp.sum(v)

    lax.fori_loop(0, n // NLANE, body, jnp.int32(0), unroll=2)

  def row_body(r, _):
    row = wid * rows_per_worker + r
    pltpu.sync_copy(x_hbm.at[row], xv)

    def valid_body(j, carry):
      fv, lv = carry
      v = xv[pl.ds(j * NLANE, NLANE)]
      nz = v != 0.0
      idxv = j * NLANE + iota
      fv = jnp.minimum(fv, jnp.where(nz, idxv, jnp.int32(L)))
      lv = jnp.maximum(lv, jnp.where(nz, idxv, jnp.int32(-1)))

      @pl.when(j < RADIX // NLANE)
      def _():
        hist0[pl.ds(j * NLANE, NLANE)] = zeros16i

      return fv, lv

    fv, lv = lax.fori_loop(0, NVREG, valid_body,
                           (zeros16i + L, zeros16i - 1), unroll=4)
    start = jnp.min(fv)
    end = jnp.max(lv)
    vlen = end - start + 1
    lenc = jnp.maximum(vlen, 1)

    def key_body(j, _):
      v = xv[pl.ds(j * NLANE, NLANE)]
      t = plsc.bitcast(v, jnp.int32)
      s = t ^ (lax.shift_right_arithmetic(t, 31) & _I32_MAX)
      u = s ^ _I32_MIN
      idxv = j * NLANE + iota
      ok = (idxv >= start) & (idxv <= end)
      key = jnp.where(ok, u, jnp.int32(-1))
      keys0[pl.ds(j * NLANE, NLANE)] = key
      pay0[pl.ds(j * NLANE, NLANE)] = idxv
      d = key & (RADIX - 1)
      occ, last = plsc.scan_count(d)
      plsc.addupdate_scatter(hist0, [d], occ, mask=last)
      return 0

    lax.fori_loop(0, NVREG, key_body, 0, unroll=4)

    exclusive_scan(hist0, hist1, RADIX, RADIX)

    def permute0_body(j, _):
      k = keys0[pl.ds(j * NLANE, NLANE)]
      p = pay0[pl.ds(j * NLANE, NLANE)]
      d = k & (RADIX - 1)
      occ, last = plsc.scan_count(d)
      base = plsc.load_gather(hist0, [d])
      slot = base + occ - 1
      plsc.store_scatter(keys1, [slot], k)
      plsc.store_scatter(pay1, [slot], p)
      plsc.addupdate_scatter(hist0, [d], occ, mask=last)
      d1 = lax.shift_right_logical(k, 11) & (RADIX - 1)
      occ1, last1 = plsc.scan_count(d1)
      plsc.addupdate_scatter(hist1, [d1], occ1, mask=last1)
      return 0

    lax.fori_loop(0, NVREG, permute0_body, 0, unroll=2)

    exclusive_scan(hist1, hist0, RADIX, 1024)

    def permute1_body(j, _):
      k = keys1[pl.ds(j * NLANE, NLANE)]
      p = pay1[pl.ds(j * NLANE, NLANE)]
      d = lax.shift_right_logical(k, 11) & (RADIX - 1)
      occ, last = plsc.scan_count(d)
      base = plsc.load_gather(hist1, [d])
      slot = base + occ - 1
      plsc.store_scatter(keys0, [slot], k)
      plsc.store_scatter(pay0, [slot], p)
      plsc.addupdate_scatter(hist1, [d], occ, mask=last)
      d2 = lax.shift_right_logical(k, 22) & 1023
      occ2, last2 = plsc.scan_count(d2)
      plsc.addupdate_scatter(hist0, [d2], occ2, mask=last2)
      return 0

    lax.fori_loop(0, NVREG, permute1_body, 0, unroll=2)

    exclusive_scan(hist0, hist1, 1024, 0)

    def hclear_body(j, _):
      rowhist[pl.ds(j * NLANE, NLANE)] = zeros16f
      return 0

    lax.fori_loop(0, HIST_PAD // NLANE, hclear_body, 0, unroll=4)

    invlen = (zeros16f + 1.0) / (zeros16i + lenc).astype(jnp.float32)

    def permute2_body(j, _):
      k = keys0[pl.ds(j * NLANE, NLANE)]
      p = pay0[pl.ds(j * NLANE, NLANE)]
      d = lax.shift_right_logical(k, 22) & 1023
      occ, last = plsc.scan_count(d)
      base = plsc.load_gather(hist0, [d])
      slot = base + occ - 1
      plsc.addupdate_scatter(hist0, [d], occ, mask=last)
      bf = (slot * NB).astype(jnp.float32) * invlen + 5e-5
      b = jnp.minimum(bf.astype(jnp.int32), NB - 1)
      plsc.store_scatter(bins, [p], b)
      return 0

    lax.fori_loop(0, NVREG, permute2_body, 0, unroll=2)

    def trans_body(j, _):
      a = bins[pl.ds(j * NLANE, NLANE)]
      b = bins[pl.ds(j * NLANE + 1, NLANE)]
      t = j * NLANE + iota
      ok = (t >= start) & (t <= end - 1)
      cell = a * NB + b
      occ, last = plsc.scan_count(cell, mask=ok)
      plsc.addupdate_scatter(rowhist, [cell], occ.astype(jnp.float32),
                             mask=last & ok)
      return 0

    lax.fori_loop(0, NVREG, trans_body, 0, unroll=4)

    denom = (zeros16i + jnp.maximum(vlen - 1, 1)).astype(jnp.float32)
    inv = (zeros16f + 1.0) / denom

    def norm_body(j, _):
      sl = pl.ds(j * NLANE, NLANE)
      rowhist[sl] = rowhist[sl] * inv
      return 0

    lax.fori_loop(0, HIST_PAD // NLANE, norm_body, 0, unroll=4)

    pltpu.sync_copy(rowhist, out_hbm.at[row])
    return 0

  lax.fori_loop(0, rows_per_worker, row_body, 0)


@jax.jit
def kernel(x):
  N, C, Lx = x.shape
  rows = N * C
  x2 = x.reshape(rows, Lx)
  mesh = plsc.VectorSubcoreMesh(core_axis_name="c", subcore_axis_name="s",
                                num_cores=NCORES, num_subcores=NSUB)
  run = functools.partial(
      pl.kernel,
      mesh=mesh,
      compiler_params=pltpu.CompilerParams(needs_layout_passes=False),
      out_type=jax.ShapeDtypeStruct((rows, HIST_PAD), jnp.float32),
      scratch_types=[
          pltpu.VMEM((L,), jnp.float32),
          pltpu.VMEM((L,), jnp.int32),
          pltpu.VMEM((L,), jnp.int32),
          pltpu.VMEM((L,), jnp.int32),
          pltpu.VMEM((L,), jnp.int32),
          pltpu.VMEM((RADIX,), jnp.int32),
          pltpu.VMEM((RADIX,), jnp.int32),
          pltpu.VMEM((L + NLANE,), jnp.int32),
          pltpu.VMEM((HIST_PAD,), jnp.float32),
      ],
  )(_row_kernel)
  out = run(x2)
  return out[:, :NB * NB].reshape(N, C, NB, NB)


# speedup vs baseline: 14.8811x; 1.0262x over previous
import functools

import jax
import jax.numpy as jnp
from jax import lax
from jax.experimental import pallas as pl
from jax.experimental.pallas import tpu as pltpu
from jax.experimental.pallas import tpu_sc as plsc

L = 4096
NB = 65
HIST_PAD = 4240
NLANE = 16
NVREG = L // NLANE
NCORES = 2
NSUB = 16
NWORKERS = NCORES * NSUB
RADIX = 1 << 11

_I32_MIN = -2147483648
_I32_MAX = 2147483647


def _row_kernel(x_hbm, out_hbm, xv, keys0, pay0, keys1, pay1, hist0, hist1,
                bins, rowhist):
  total_rows = x_hbm.shape[0]
  rows_per_worker = total_rows // NWORKERS
  wid = lax.axis_index("s") * NCORES + lax.axis_index("c")
  iota = lax.iota(jnp.int32, NLANE)
  zeros16i = jnp.zeros((NLANE,), jnp.int32)
  zeros16f = jnp.zeros((NLANE,), jnp.float32)

  bins[pl.ds(L, NLANE)] = zeros16i

  def exclusive_scan(src, dst_clear, n, m):
    def body(j, carry):
      v = src[pl.ds(j * NLANE, NLANE)]
      inc = plsc.cumsum(v)
      src[pl.ds(j * NLANE, NLANE)] = inc - v + carry

      @pl.when(j < m // NLANE)
      def _():
        dst_clear[pl.ds(j * NLANE, NLANE)] = zeros16i

      return carry + jnp.sum(v)

    lax.fori_loop(0, n // NLANE, body, jnp.int32(0), unroll=2)

  def row_body(r, _):
    row = wid * rows_per_worker + r
    pltpu.sync_copy(x_hbm.at[row], xv)

    def h0clear_body(j, _):
      hist0[pl.ds(j * NLANE, NLANE)] = zeros16i
      return 0

    lax.fori_loop(0, RADIX // NLANE, h0clear_body, 0, unroll=8)

    def keyfast_body(j, zacc):
      v = xv[pl.ds(j * NLANE, NLANE)]
      t = plsc.bitcast(v, jnp.int32)
      s = t ^ (lax.shift_right_arithmetic(t, 31) & _I32_MAX)
      u = s ^ _I32_MIN
      idxv = j * NLANE + iota
      keys0[pl.ds(j * NLANE, NLANE)] = u
      pay0[pl.ds(j * NLANE, NLANE)] = idxv
      d = u & (RADIX - 1)
      occ, last = plsc.scan_count(d)
      plsc.addupdate_scatter(hist0, [d], occ, mask=last)
      return zacc | (t + t == 0)

    zacc = lax.fori_loop(0, NVREG, keyfast_body, iota < 0, unroll=4)
    anyzero = jnp.max(zacc.astype(jnp.int32)) > 0

    def slow_path():
      def valid_body(j, carry):
        fv, lv = carry
        v = xv[pl.ds(j * NLANE, NLANE)]
        nz = v != 0.0
        idxv = j * NLANE + iota
        fv = jnp.minimum(fv, jnp.where(nz, idxv, jnp.int32(L)))
        lv = jnp.maximum(lv, jnp.where(nz, idxv, jnp.int32(-1)))
        return fv, lv

      fv, lv = lax.fori_loop(0, NVREG, valid_body,
                             (zeros16i + L, zeros16i - 1), unroll=4)
      s_, e_ = jnp.min(fv), jnp.max(lv)

      def hclear(j, _):
        hist0[pl.ds(j * NLANE, NLANE)] = zeros16i
        return 0

      lax.fori_loop(0, RADIX // NLANE, hclear, 0, unroll=4)

      def keymask_body(j, _):
        u = keys0[pl.ds(j * NLANE, NLANE)]
        idxv = j * NLANE + iota
        ok = (idxv >= s_) & (idxv <= e_)
        key = jnp.where(ok, u, jnp.int32(-1))
        keys0[pl.ds(j * NLANE, NLANE)] = key
        d = key & (RADIX - 1)
        occ, last = plsc.scan_count(d)
        plsc.addupdate_scatter(hist0, [d], occ, mask=last)
        return 0

      lax.fori_loop(0, NVREG, keymask_body, 0, unroll=4)
      return s_, e_

    start, end = lax.cond(anyzero, slow_path,
                          lambda: (jnp.int32(0), jnp.int32(L - 1)))
    vlen = end - start + 1
    lenc = jnp.maximum(vlen, 1)

    exclusive_scan(hist0, hist1, RADIX, RADIX)

    def permute0_body(j, _):
      k = keys0[pl.ds(j * NLANE, NLANE)]
      p = pay0[pl.ds(j * NLANE, NLANE)]
      d = k & (RADIX - 1)
      occ, last = plsc.scan_count(d)
      base = plsc.load_gather(hist0, [d])
      slot = base + occ - 1
      plsc.store_scatter(keys1, [slot], k)
      plsc.store_scatter(pay1, [slot], p)
      plsc.addupdate_scatter(hist0, [d], occ, mask=last)
      d1 = lax.shift_right_logical(k, 11) & (RADIX - 1)
      occ1, last1 = plsc.scan_count(d1)
      plsc.addupdate_scatter(hist1, [d1], occ1, mask=last1)
      return 0

    lax.fori_loop(0, NVREG, permute0_body, 0, unroll=4)

    exclusive_scan(hist1, hist0, RADIX, 1024)

    def permute1_body(j, _):
      k = keys1[pl.ds(j * NLANE, NLANE)]
      p = pay1[pl.ds(j * NLANE, NLANE)]
      d = lax.shift_right_logical(k, 11) & (RADIX - 1)
      occ, last = plsc.scan_count(d)
      base = plsc.load_gather(hist1, [d])
      slot = base + occ - 1
      plsc.store_scatter(keys0, [slot], k)
      plsc.store_scatter(pay0, [slot], p)
      plsc.addupdate_scatter(hist1, [d], occ, mask=last)
      d2 = lax.shift_right_logical(k, 22) & 1023
      occ2, last2 = plsc.scan_count(d2)
      plsc.addupdate_scatter(hist0, [d2], occ2, mask=last2)
      return 0

    lax.fori_loop(0, NVREG, permute1_body, 0, unroll=4)

    exclusive_scan(hist0, hist1, 1024, 0)

    def hclear_body(j, _):
      rowhist[pl.ds(j * NLANE, NLANE)] = zeros16f
      return 0

    lax.fori_loop(0, HIST_PAD // NLANE, hclear_body, 0, unroll=4)

    invlen = (zeros16f + 1.0) / (zeros16i + lenc).astype(jnp.float32)

    def permute2_body(j, _):
      k = keys0[pl.ds(j * NLANE, NLANE)]
      p = pay0[pl.ds(j * NLANE, NLANE)]
      d = lax.shift_right_logical(k, 22) & 1023
      occ, last = plsc.scan_count(d)
      base = plsc.load_gather(hist0, [d])
      slot = base + occ - 1
      plsc.addupdate_scatter(hist0, [d], occ, mask=last)
      bf = (slot * NB).astype(jnp.float32) * invlen + 5e-5
      b = jnp.minimum(bf.astype(jnp.int32), NB - 1)
      plsc.store_scatter(bins, [p], b)
      return 0

    lax.fori_loop(0, NVREG, permute2_body, 0, unroll=4)

    def trans_masked(j, _):
      a = bins[pl.ds(j * NLANE, NLANE)]
      b = bins[pl.ds(j * NLANE + 1, NLANE)]
      t = j * NLANE + iota
      ok = (t >= start) & (t <= end - 1)
      cell = a * NB + b
      occ, last = plsc.scan_count(cell, mask=ok)
      plsc.addupdate_scatter(rowhist, [cell], occ.astype(jnp.float32),
                             mask=last & ok)
      return 0

    def trans_fast(j, _):
      a = bins[pl.ds(j * NLANE, NLANE)]
      b = bins[pl.ds(j * NLANE + 1, NLANE)]
      cell = a * NB + b
      occ, last = plsc.scan_count(cell)
      plsc.addupdate_scatter(rowhist, [cell], occ.astype(jnp.float32),
                             mask=last)
      return 0

    def trans_all_masked():
      lax.fori_loop(0, NVREG, trans_masked, 0, unroll=4)
      return 0

    def trans_all_fast():
      lax.fori_loop(0, NVREG - 1, trans_fast, 0, unroll=4)
      trans_masked(NVREG - 1, 0)
      return 0

    lax.cond(anyzero, trans_all_masked, trans_all_fast)

    denom = (zeros16i + jnp.maximum(vlen - 1, 1)).astype(jnp.float32)
    inv = (zeros16f + 1.0) / denom

    def norm_body(j, _):
      sl = pl.ds(j * NLANE, NLANE)
      rowhist[sl] = rowhist[sl] * inv
      return 0

    lax.fori_loop(0, HIST_PAD // NLANE, norm_body, 0, unroll=4)

    pltpu.sync_copy(rowhist, out_hbm.at[row])
    return 0

  lax.fori_loop(0, rows_per_worker, row_body, 0)


@jax.jit
def kernel(x):
  N, C, Lx = x.shape
  rows = N * C
  x2 = x.reshape(rows, Lx)
  mesh = plsc.VectorSubcoreMesh(core_axis_name="c", subcore_axis_name="s",
                                num_cores=NCORES, num_subcores=NSUB)
  run = functools.partial(
      pl.kernel,
      mesh=mesh,
      compiler_params=pltpu.CompilerParams(needs_layout_passes=False),
      out_type=jax.ShapeDtypeStruct((rows, HIST_PAD), jnp.float32),
      scratch_types=[
          pltpu.VMEM((L,), jnp.float32),
          pltpu.VMEM((L,), jnp.int32),
          pltpu.VMEM((L,), jnp.int32),
          pltpu.VMEM((L,), jnp.int32),
          pltpu.VMEM((L,), jnp.int32),
          pltpu.VMEM((RADIX,), jnp.int32),
          pltpu.VMEM((RADIX,), jnp.int32),
          pltpu.VMEM((L + NLANE,), jnp.int32),
          pltpu.VMEM((HIST_PAD,), jnp.float32),
      ],
  )(_row_kernel)
  out = run(x2)
  return out[:, :NB * NB].reshape(N, C, NB, NB)


# speedup vs baseline: 15.2983x; 1.0280x over previous
import functools

import jax
import jax.numpy as jnp
from jax import lax
from jax.experimental import pallas as pl
from jax.experimental.pallas import tpu as pltpu
from jax.experimental.pallas import tpu_sc as plsc

L = 4096
NB = 65
HIST_PAD = 4240
NLANE = 16
NVREG = L // NLANE
NCORES = 2
NSUB = 16
NWORKERS = NCORES * NSUB
RADIX = 1 << 11

_I32_MIN = -2147483648
_I32_MAX = 2147483647


def _row_kernel(x_hbm, out_hbm,
                xvA, keys0A, pay0A, keys1A, pay1A, hist0A, hist1A, binsA,
                rowhistA,
                xvB, keys0B, pay0B, keys1B, pay1B, hist0B, hist1B, binsB,
                rowhistB,
                semA, semB, osemA, osemB):
  total_rows = x_hbm.shape[0]
  rows_per_worker = total_rows // NWORKERS
  npairs = rows_per_worker // 2
  wid = lax.axis_index("s") * NCORES + lax.axis_index("c")
  iota = lax.iota(jnp.int32, NLANE)
  zeros16i = jnp.zeros((NLANE,), jnp.int32)
  zeros16f = jnp.zeros((NLANE,), jnp.float32)

  binsA[pl.ds(L, NLANE)] = zeros16i
  binsB[pl.ds(L, NLANE)] = zeros16i

  def pair_body(r, _):
    rowA = wid * rows_per_worker + 2 * r
    rowB = rowA + 1

    @pl.when(r > 0)
    def _():
      pltpu.make_async_copy(rowhistA, out_hbm.at[rowA - 2], osemA).wait()
      pltpu.make_async_copy(rowhistB, out_hbm.at[rowB - 2], osemB).wait()

    cpA = pltpu.make_async_copy(x_hbm.at[rowA], xvA, semA)
    cpB = pltpu.make_async_copy(x_hbm.at[rowB], xvB, semB)
    cpA.start()
    cpB.start()
    cpA.wait()
    cpB.wait()

    def h0clear_body(j, _):
      hist0A[pl.ds(j * NLANE, NLANE)] = zeros16i
      hist0B[pl.ds(j * NLANE, NLANE)] = zeros16i
      return 0

    lax.fori_loop(0, RADIX // NLANE, h0clear_body, 0, unroll=4)

    def key_one(j, xv, keys0, pay0, hist0, zacc):
      v = xv[pl.ds(j * NLANE, NLANE)]
      t = plsc.bitcast(v, jnp.int32)
      s = t ^ (lax.shift_right_arithmetic(t, 31) & _I32_MAX)
      u = s ^ _I32_MIN
      idxv = j * NLANE + iota
      keys0[pl.ds(j * NLANE, NLANE)] = u
      pay0[pl.ds(j * NLANE, NLANE)] = idxv
      d = u & (RADIX - 1)
      occ, last = plsc.scan_count(d)
      plsc.addupdate_scatter(hist0, [d], occ, mask=last)
      return zacc | (t + t == 0)

    def keyfast_body(j, carry):
      zA, zB = carry
      zA = key_one(j, xvA, keys0A, pay0A, hist0A, zA)
      zB = key_one(j, xvB, keys0B, pay0B, hist0B, zB)
      return zA, zB

    zA, zB = lax.fori_loop(0, NVREG, keyfast_body, (iota < 0, iota < 0),
                           unroll=2)

    def make_slow_path(xv, keys0, hist0):
      def slow_path():
        def valid_body(j, carry):
          fv, lv = carry
          v = xv[pl.ds(j * NLANE, NLANE)]
          nz = v != 0.0
          idxv = j * NLANE + iota
          fv = jnp.minimum(fv, jnp.where(nz, idxv, jnp.int32(L)))
          lv = jnp.maximum(lv, jnp.where(nz, idxv, jnp.int32(-1)))
          return fv, lv

        fv, lv = lax.fori_loop(0, NVREG, valid_body,
                               (zeros16i + L, zeros16i - 1), unroll=4)
        s_, e_ = jnp.min(fv), jnp.max(lv)

        def hclear(j, _):
          hist0[pl.ds(j * NLANE, NLANE)] = zeros16i
          return 0

        lax.fori_loop(0, RADIX // NLANE, hclear, 0, unroll=4)

        def keymask_body(j, _):
          u = keys0[pl.ds(j * NLANE, NLANE)]
          idxv = j * NLANE + iota
          ok = (idxv >= s_) & (idxv <= e_)
          key = jnp.where(ok, u, jnp.int32(-1))
          keys0[pl.ds(j * NLANE, NLANE)] = key
          d = key & (RADIX - 1)
          occ, last = plsc.scan_count(d)
          plsc.addupdate_scatter(hist0, [d], occ, mask=last)
          return 0

        lax.fori_loop(0, NVREG, keymask_body, 0, unroll=4)
        return s_, e_

      return slow_path

    full = lambda: (jnp.int32(0), jnp.int32(L - 1))
    anyzeroA = jnp.max(zA.astype(jnp.int32)) > 0
    anyzeroB = jnp.max(zB.astype(jnp.int32)) > 0
    startA, endA = lax.cond(anyzeroA, make_slow_path(xvA, keys0A, hist0A),
                            full)
    startB, endB = lax.cond(anyzeroB, make_slow_path(xvB, keys0B, hist0B),
                            full)
    vlenA = endA - startA + 1
    vlenB = endB - startB + 1
    lencA = jnp.maximum(vlenA, 1)
    lencB = jnp.maximum(vlenB, 1)

    def exclusive_scan2(srcA, srcB, clrA, clrB, n, m):
      def body(j, carry):
        cA, cB = carry
        vA = srcA[pl.ds(j * NLANE, NLANE)]
        vB = srcB[pl.ds(j * NLANE, NLANE)]
        incA = plsc.cumsum(vA)
        incB = plsc.cumsum(vB)
        srcA[pl.ds(j * NLANE, NLANE)] = incA - vA + cA
        srcB[pl.ds(j * NLANE, NLANE)] = incB - vB + cB

        @pl.when(j < m // NLANE)
        def _():
          clrA[pl.ds(j * NLANE, NLANE)] = zeros16i
          clrB[pl.ds(j * NLANE, NLANE)] = zeros16i

        return cA + jnp.sum(vA), cB + jnp.sum(vB)

      lax.fori_loop(0, n // NLANE, body, (jnp.int32(0), jnp.int32(0)),
                    unroll=2)

    exclusive_scan2(hist0A, hist0B, hist1A, hist1B, RADIX, RADIX)

    def permute01_one(j, sh, kin, pin, kout, pout, hist, histnext, nbits2):
      k = kin[pl.ds(j * NLANE, NLANE)]
      p = pin[pl.ds(j * NLANE, NLANE)]
      d = lax.shift_right_logical(k, sh) & (RADIX - 1) if sh else \
          k & (RADIX - 1)
      occ, last = plsc.scan_count(d)
      base = plsc.load_gather(hist, [d])
      slot = base + occ - 1
      plsc.store_scatter(kout, [slot], k)
      plsc.store_scatter(pout, [slot], p)
      plsc.addupdate_scatter(hist, [d], occ, mask=last)
      dn = lax.shift_right_logical(k, sh + 11) & ((1 << nbits2) - 1)
      occn, lastn = plsc.scan_count(dn)
      plsc.addupdate_scatter(histnext, [dn], occn, mask=lastn)
      return 0

    def permute0_body(j, _):
      permute01_one(j, 0, keys0A, pay0A, keys1A, pay1A, hist0A, hist1A, 11)
      permute01_one(j, 0, keys0B, pay0B, keys1B, pay1B, hist0B, hist1B, 11)
      return 0

    lax.fori_loop(0, NVREG, permute0_body, 0, unroll=2)

    exclusive_scan2(hist1A, hist1B, hist0A, hist0B, RADIX, 1024)

    def permute1_body(j, _):
      permute01_one(j, 11, keys1A, pay1A, keys0A, pay0A, hist1A, hist0A, 10)
      permute01_one(j, 11, keys1B, pay1B, keys0B, pay0B, hist1B, hist0B, 10)
      return 0

    lax.fori_loop(0, NVREG, permute1_body, 0, unroll=2)

    exclusive_scan2(hist0A, hist0B, hist1A, hist1B, 1024, 0)

    def hclear_body(j, _):
      rowhistA[pl.ds(j * NLANE, NLANE)] = zeros16f
      rowhistB[pl.ds(j * NLANE, NLANE)] = zeros16f
      return 0

    lax.fori_loop(0, HIST_PAD // NLANE, hclear_body, 0, unroll=4)

    invlenA = (zeros16f + 1.0) / (zeros16i + lencA).astype(jnp.float32)
    invlenB = (zeros16f + 1.0) / (zeros16i + lencB).astype(jnp.float32)

    def permute2_one(j, keys0, pay0, hist0, bins, invlen):
      k = keys0[pl.ds(j * NLANE, NLANE)]
      p = pay0[pl.ds(j * NLANE, NLANE)]
      d = lax.shift_right_logical(k, 22) & 1023
      occ, last = plsc.scan_count(d)
      base = plsc.load_gather(hist0, [d])
      slot = base + occ - 1
      plsc.addupdate_scatter(hist0, [d], occ, mask=last)
      bf = (slot * NB).astype(jnp.float32) * invlen + 5e-5
      b = jnp.minimum(bf.astype(jnp.int32), NB - 1)
      plsc.store_scatter(bins, [p], b)
      return 0

    def permute2_body(j, _):
      permute2_one(j, keys0A, pay0A, hist0A, binsA, invlenA)
      permute2_one(j, keys0B, pay0B, hist0B, binsB, invlenB)
      return 0

    lax.fori_loop(0, NVREG, permute2_body, 0, unroll=2)

    def trans_masked_one(j, bins, rowhist, start, end):
      a = bins[pl.ds(j * NLANE, NLANE)]
      b = bins[pl.ds(j * NLANE + 1, NLANE)]
      t = j * NLANE + iota
      ok = (t >= start) & (t <= end - 1)
      cell = a * NB + b
      occ, last = plsc.scan_count(cell, mask=ok)
      plsc.addupdate_scatter(rowhist, [cell], occ.astype(jnp.float32),
                             mask=last & ok)
      return 0

    def trans_fast_one(j, bins, rowhist):
      a = bins[pl.ds(j * NLANE, NLANE)]
      b = bins[pl.ds(j * NLANE + 1, NLANE)]
      cell = a * NB + b
      occ, last = plsc.scan_count(cell)
      plsc.addupdate_scatter(rowhist, [cell], occ.astype(jnp.float32),
                             mask=last)
      return 0

    def trans_all_masked():
      def body(j, _):
        trans_masked_one(j, binsA, rowhistA, startA, endA)
        trans_masked_one(j, binsB, rowhistB, startB, endB)
        return 0

      lax.fori_loop(0, NVREG, body, 0, unroll=2)
      return 0

    def trans_all_fast():
      def body(j, _):
        trans_fast_one(j, binsA, rowhistA)
        trans_fast_one(j, binsB, rowhistB)
        return 0

      lax.fori_loop(0, NVREG - 1, body, 0, unroll=2)
      trans_masked_one(NVREG - 1, binsA, rowhistA, startA, endA)
      trans_masked_one(NVREG - 1, binsB, rowhistB, startB, endB)
      return 0

    lax.cond(anyzeroA | anyzeroB, trans_all_masked, trans_all_fast)

    invA = (zeros16f + 1.0) / \
        (zeros16i + jnp.maximum(vlenA - 1, 1)).astype(jnp.float32)
    invB = (zeros16f + 1.0) / \
        (zeros16i + jnp.maximum(vlenB - 1, 1)).astype(jnp.float32)

    def norm_body(j, _):
      sl = pl.ds(j * NLANE, NLANE)
      rowhistA[sl] = rowhistA[sl] * invA
      rowhistB[sl] = rowhistB[sl] * invB
      return 0

    lax.fori_loop(0, HIST_PAD // NLANE, norm_body, 0, unroll=4)

    pltpu.make_async_copy(rowhistA, out_hbm.at[rowA], osemA).start()
    pltpu.make_async_copy(rowhistB, out_hbm.at[rowB], osemB).start()
    return 0

  lax.fori_loop(0, npairs, pair_body, 0)
  last_rowA = wid * rows_per_worker + 2 * (npairs - 1)
  pltpu.make_async_copy(rowhistA, out_hbm.at[last_rowA], osemA).wait()
  pltpu.make_async_copy(rowhistB, out_hbm.at[last_rowA + 1], osemB).wait()


@jax.jit
def kernel(x):
  N, C, Lx = x.shape
  rows = N * C
  x2 = x.reshape(rows, Lx)
  mesh = plsc.VectorSubcoreMesh(core_axis_name="c", subcore_axis_name="s",
                                num_cores=NCORES, num_subcores=NSUB)
  per_row_scratch = [
      pltpu.VMEM((L,), jnp.float32),
      pltpu.VMEM((L,), jnp.int32),
      pltpu.VMEM((L,), jnp.int32),
      pltpu.VMEM((L,), jnp.int32),
      pltpu.VMEM((L,), jnp.int32),
      pltpu.VMEM((RADIX,), jnp.int32),
      pltpu.VMEM((RADIX,), jnp.int32),
      pltpu.VMEM((L + NLANE,), jnp.int32),
      pltpu.VMEM((HIST_PAD,), jnp.float32),
  ]
  run = functools.partial(
      pl.kernel,
      mesh=mesh,
      compiler_params=pltpu.CompilerParams(needs_layout_passes=False),
      out_type=jax.ShapeDtypeStruct((rows, HIST_PAD), jnp.float32),
      scratch_types=per_row_scratch + per_row_scratch + [
          pltpu.SemaphoreType.DMA,
          pltpu.SemaphoreType.DMA,
          pltpu.SemaphoreType.DMA,
          pltpu.SemaphoreType.DMA,
      ],
  )(_row_kernel)
  out = run(x2)
  return out[:, :NB * NB].reshape(N, C, NB, NB)


# speedup vs baseline: 23.8932x; 1.5618x over previous
import functools

import jax
import jax.numpy as jnp
from jax import lax
from jax.experimental import pallas as pl
from jax.experimental.pallas import tpu as pltpu
from jax.experimental.pallas import tpu_sc as plsc

L = 4096
NB = 65
HIST_PAD = 4240
NLANE = 16
NVREG = L // NLANE
NCORES = 2
NSUB = 16
NWORKERS = NCORES * NSUB
RADIX = 1 << 11

_I32_MIN = -2147483648
_I32_MAX = 2147483647


def _row_kernel(x_hbm, out_hbm,
                xvA, keys0A, pay0A, keys1A, pay1A, hist0A, hist1A, binsA,
                rowhistA,
                xvB, keys0B, pay0B, keys1B, pay1B, hist0B, hist1B, binsB,
                rowhistB,
                semA, semB, osemA, osemB):
  total_rows = x_hbm.shape[0]
  rows_per_worker = total_rows // NWORKERS
  npairs = rows_per_worker // 2
  wid = lax.axis_index("s") * NCORES + lax.axis_index("c")
  iota = lax.iota(jnp.int32, NLANE)
  zeros16i = jnp.zeros((NLANE,), jnp.int32)
  zeros16f = jnp.zeros((NLANE,), jnp.float32)

  binsA[pl.ds(L, NLANE)] = zeros16i
  binsB[pl.ds(L, NLANE)] = zeros16i

  def pair_body(r, _):
    rowA = wid * rows_per_worker + 2 * r
    rowB = rowA + 1

    @pl.when(r > 0)
    def _():
      pltpu.make_async_copy(rowhistA, out_hbm.at[rowA - 2], osemA).wait()
      pltpu.make_async_copy(rowhistB, out_hbm.at[rowB - 2], osemB).wait()

    cpA = pltpu.make_async_copy(x_hbm.at[rowA], xvA, semA)
    cpB = pltpu.make_async_copy(x_hbm.at[rowB], xvB, semB)
    cpA.start()
    cpB.start()
    cpA.wait()
    cpB.wait()

    def h0clear_body(j, _):
      hist0A[pl.ds(j * NLANE, NLANE)] = zeros16i
      hist0B[pl.ds(j * NLANE, NLANE)] = zeros16i
      return 0

    lax.fori_loop(0, RADIX // NLANE, h0clear_body, 0, unroll=4)

    def keyfast_body(j, carry):
      zA, zB = carry
      idxv = j * NLANE + iota
      vA = xvA[pl.ds(j * NLANE, NLANE)]
      vB = xvB[pl.ds(j * NLANE, NLANE)]
      tA = plsc.bitcast(vA, jnp.int32)
      tB = plsc.bitcast(vB, jnp.int32)
      uA = (tA ^ (lax.shift_right_arithmetic(tA, 31) & _I32_MAX)) ^ _I32_MIN
      uB = (tB ^ (lax.shift_right_arithmetic(tB, 31) & _I32_MAX)) ^ _I32_MIN
      dA = uA & (RADIX - 1)
      dB = uB & (RADIX - 1)
      occA, lastA = plsc.scan_count(dA)
      occB, lastB = plsc.scan_count(dB)
      keys0A[pl.ds(j * NLANE, NLANE)] = uA
      keys0B[pl.ds(j * NLANE, NLANE)] = uB
      pay0A[pl.ds(j * NLANE, NLANE)] = idxv
      pay0B[pl.ds(j * NLANE, NLANE)] = idxv
      plsc.addupdate_scatter(hist0A, [dA], occA, mask=lastA)
      plsc.addupdate_scatter(hist0B, [dB], occB, mask=lastB)
      return zA | (tA + tA == 0), zB | (tB + tB == 0)

    zA, zB = lax.fori_loop(0, NVREG, keyfast_body, (iota < 0, iota < 0),
                           unroll=2)

    def make_slow_path(xv, keys0, hist0):
      def slow_path():
        def valid_body(j, carry):
          fv, lv = carry
          v = xv[pl.ds(j * NLANE, NLANE)]
          nz = v != 0.0
          idxv = j * NLANE + iota
          fv = jnp.minimum(fv, jnp.where(nz, idxv, jnp.int32(L)))
          lv = jnp.maximum(lv, jnp.where(nz, idxv, jnp.int32(-1)))
          return fv, lv

        fv, lv = lax.fori_loop(0, NVREG, valid_body,
                               (zeros16i + L, zeros16i - 1), unroll=4)
        s_, e_ = jnp.min(fv), jnp.max(lv)

        def hclear(j, _):
          hist0[pl.ds(j * NLANE, NLANE)] = zeros16i
          return 0

        lax.fori_loop(0, RADIX // NLANE, hclear, 0, unroll=4)

        def keymask_body(j, _):
          u = keys0[pl.ds(j * NLANE, NLANE)]
          idxv = j * NLANE + iota
          ok = (idxv >= s_) & (idxv <= e_)
          key = jnp.where(ok, u, jnp.int32(-1))
          keys0[pl.ds(j * NLANE, NLANE)] = key
          d = key & (RADIX - 1)
          occ, last = plsc.scan_count(d)
          plsc.addupdate_scatter(hist0, [d], occ, mask=last)
          return 0

        lax.fori_loop(0, NVREG, keymask_body, 0, unroll=4)
        return s_, e_

      return slow_path

    full = lambda: (jnp.int32(0), jnp.int32(L - 1))
    anyzeroA = jnp.max(zA.astype(jnp.int32)) > 0
    anyzeroB = jnp.max(zB.astype(jnp.int32)) > 0
    startA, endA = lax.cond(anyzeroA, make_slow_path(xvA, keys0A, hist0A),
                            full)
    startB, endB = lax.cond(anyzeroB, make_slow_path(xvB, keys0B, hist0B),
                            full)
    vlenA = endA - startA + 1
    vlenB = endB - startB + 1
    lencA = jnp.maximum(vlenA, 1)
    lencB = jnp.maximum(vlenB, 1)

    def exclusive_scan2(srcA, srcB, clrA, clrB, n, m):
      def body(j, carry):
        cA, cB = carry
        vA = srcA[pl.ds(j * NLANE, NLANE)]
        vB = srcB[pl.ds(j * NLANE, NLANE)]
        incA = plsc.cumsum(vA)
        incB = plsc.cumsum(vB)
        srcA[pl.ds(j * NLANE, NLANE)] = incA - vA + cA
        srcB[pl.ds(j * NLANE, NLANE)] = incB - vB + cB

        @pl.when(j < m // NLANE)
        def _():
          clrA[pl.ds(j * NLANE, NLANE)] = zeros16i
          clrB[pl.ds(j * NLANE, NLANE)] = zeros16i

        return cA + jnp.sum(vA), cB + jnp.sum(vB)

      lax.fori_loop(0, n // NLANE, body, (jnp.int32(0), jnp.int32(0)),
                    unroll=2)

    exclusive_scan2(hist0A, hist0B, hist1A, hist1B, RADIX, RADIX)

    def permute01_pair(j, sh, nbits2, Ar, Br):
      kinA, pinA, koutA, poutA, histA, histnextA = Ar
      kinB, pinB, koutB, poutB, histB, histnextB = Br
      sl = pl.ds(j * NLANE, NLANE)
      kA = kinA[sl]
      kB = kinB[sl]
      pA = pinA[sl]
      pB = pinB[sl]
      dA = lax.shift_right_logical(kA, sh) & (RADIX - 1)
      dB = lax.shift_right_logical(kB, sh) & (RADIX - 1)
      dnA = lax.shift_right_logical(kA, sh + 11) & ((1 << nbits2) - 1)
      dnB = lax.shift_right_logical(kB, sh + 11) & ((1 << nbits2) - 1)
      occA, lastA = plsc.scan_count(dA)
      occB, lastB = plsc.scan_count(dB)
      occnA, lastnA = plsc.scan_count(dnA)
      occnB, lastnB = plsc.scan_count(dnB)
      baseA = plsc.load_gather(histA, [dA])
      baseB = plsc.load_gather(histB, [dB])
      slotA = baseA + occA - 1
      slotB = baseB + occB - 1
      plsc.store_scatter(koutA, [slotA], kA)
      plsc.store_scatter(koutB, [slotB], kB)
      plsc.store_scatter(poutA, [slotA], pA)
      plsc.store_scatter(poutB, [slotB], pB)
      plsc.addupdate_scatter(histA, [dA], occA, mask=lastA)
      plsc.addupdate_scatter(histB, [dB], occB, mask=lastB)
      plsc.addupdate_scatter(histnextA, [dnA], occnA, mask=lastnA)
      plsc.addupdate_scatter(histnextB, [dnB], occnB, mask=lastnB)
      return 0

    def permute0_body(j, _):
      return permute01_pair(
          j, 0, 11,
          (keys0A, pay0A, keys1A, pay1A, hist0A, hist1A),
          (keys0B, pay0B, keys1B, pay1B, hist0B, hist1B))

    lax.fori_loop(0, NVREG, permute0_body, 0, unroll=2)

    exclusive_scan2(hist1A, hist1B, hist0A, hist0B, RADIX, 1024)

    def permute1_body(j, _):
      return permute01_pair(
          j, 11, 10,
          (keys1A, pay1A, keys0A, pay0A, hist1A, hist0A),
          (keys1B, pay1B, keys0B, pay0B, hist1B, hist0B))

    lax.fori_loop(0, NVREG, permute1_body, 0, unroll=2)

    exclusive_scan2(hist0A, hist0B, hist1A, hist1B, 1024, 0)

    def hclear_body(j, _):
      rowhistA[pl.ds(j * NLANE, NLANE)] = zeros16f
      rowhistB[pl.ds(j * NLANE, NLANE)] = zeros16f
      return 0

    lax.fori_loop(0, HIST_PAD // NLANE, hclear_body, 0, unroll=4)

    invlenA = (zeros16f + 1.0) / (zeros16i + lencA).astype(jnp.float32)
    invlenB = (zeros16f + 1.0) / (zeros16i + lencB).astype(jnp.float32)

    def permute2_body(j, _):
      sl = pl.ds(j * NLANE, NLANE)
      kA = keys0A[sl]
      kB = keys0B[sl]
      pA = pay0A[sl]
      pB = pay0B[sl]
      dA = lax.shift_right_logical(kA, 22) & 1023
      dB = lax.shift_right_logical(kB, 22) & 1023
      occA, lastA = plsc.scan_count(dA)
      occB, lastB = plsc.scan_count(dB)
      baseA = plsc.load_gather(hist0A, [dA])
      baseB = plsc.load_gather(hist0B, [dB])
      slotA = baseA + occA - 1
      slotB = baseB + occB - 1
      plsc.addupdate_scatter(hist0A, [dA], occA, mask=lastA)
      plsc.addupdate_scatter(hist0B, [dB], occB, mask=lastB)
      bfA = (slotA * NB).astype(jnp.float32) * invlenA + 5e-5
      bfB = (slotB * NB).astype(jnp.float32) * invlenB + 5e-5
      bA = jnp.minimum(bfA.astype(jnp.int32), NB - 1)
      bB = jnp.minimum(bfB.astype(jnp.int32), NB - 1)
      plsc.store_scatter(binsA, [pA], bA)
      plsc.store_scatter(binsB, [pB], bB)
      return 0

    lax.fori_loop(0, NVREG, permute2_body, 0, unroll=2)

    def trans_masked_one(j, bins, rowhist, start, end):
      a = bins[pl.ds(j * NLANE, NLANE)]
      b = bins[pl.ds(j * NLANE + 1, NLANE)]
      t = j * NLANE + iota
      ok = (t >= start) & (t <= end - 1)
      cell = a * NB + b
      occ, last = plsc.scan_count(cell, mask=ok)
      plsc.addupdate_scatter(rowhist, [cell], occ.astype(jnp.float32),
                             mask=last & ok)
      return 0

    def trans_fast_one(j, bins, rowhist):
      a = bins[pl.ds(j * NLANE, NLANE)]
      b = bins[pl.ds(j * NLANE + 1, NLANE)]
      cell = a * NB + b
      occ, last = plsc.scan_count(cell)
      plsc.addupdate_scatter(rowhist, [cell], occ.astype(jnp.float32),
                             mask=last)
      return 0

    def trans_all_masked():
      def body(j, _):
        trans_masked_one(j, binsA, rowhistA, startA, endA)
        trans_masked_one(j, binsB, rowhistB, startB, endB)
        return 0

      lax.fori_loop(0, NVREG, body, 0, unroll=2)
      return 0

    def trans_all_fast():
      def body(j, _):
        sl = pl.ds(j * NLANE, NLANE)
        sl1 = pl.ds(j * NLANE + 1, NLANE)
        aA = binsA[sl]
        aB = binsB[sl]
        bA = binsA[sl1]
        bB = binsB[sl1]
        cellA = aA * NB + bA
        cellB = aB * NB + bB
        occA, lastA = plsc.scan_count(cellA)
        occB, lastB = plsc.scan_count(cellB)
        plsc.addupdate_scatter(rowhistA, [cellA], occA.astype(jnp.float32),
                               mask=lastA)
        plsc.addupdate_scatter(rowhistB, [cellB], occB.astype(jnp.float32),
                               mask=lastB)
        return 0

      lax.fori_loop(0, NVREG - 1, body, 0, unroll=2)
      trans_masked_one(NVREG - 1, binsA, rowhistA, startA, endA)
      trans_masked_one(NVREG - 1, binsB, rowhistB, startB, endB)
      return 0

    lax.cond(anyzeroA | anyzeroB, trans_all_masked, trans_all_fast)

    invA = (zeros16f + 1.0) / \
        (zeros16i + jnp.maximum(vlenA - 1, 1)).astype(jnp.float32)
    invB = (zeros16f + 1.0) / \
        (zeros16i + jnp.maximum(vlenB - 1, 1)).astype(jnp.float32)

    def norm_body(j, _):
      sl = pl.ds(j * NLANE, NLANE)
      rowhistA[sl] = rowhistA[sl] * invA
      rowhistB[sl] = rowhistB[sl] * invB
      return 0

    lax.fori_loop(0, HIST_PAD // NLANE, norm_body, 0, unroll=4)

    pltpu.make_async_copy(rowhistA, out_hbm.at[rowA], osemA).start()
    pltpu.make_async_copy(rowhistB, out_hbm.at[rowB], osemB).start()
    return 0

  lax.fori_loop(0, npairs, pair_body, 0)
  last_rowA = wid * rows_per_worker + 2 * (npairs - 1)
  pltpu.make_async_copy(rowhistA, out_hbm.at[last_rowA], osemA).wait()
  pltpu.make_async_copy(rowhistB, out_hbm.at[last_rowA + 1], osemB).wait()


@jax.jit
def kernel(x):
  N, C, Lx = x.shape
  rows = N * C
  x2 = x.reshape(rows, Lx)
  mesh = plsc.VectorSubcoreMesh(core_axis_name="c", subcore_axis_name="s",
                                num_cores=NCORES, num_subcores=NSUB)
  per_row_scratch = [
      pltpu.VMEM((L,), jnp.float32),
      pltpu.VMEM((L,), jnp.int32),
      pltpu.VMEM((L,), jnp.int32),
      pltpu.VMEM((L,), jnp.int32),
      pltpu.VMEM((L,), jnp.int32),
      pltpu.VMEM((RADIX,), jnp.int32),
      pltpu.VMEM((RADIX,), jnp.int32),
      pltpu.VMEM((L + NLANE,), jnp.int32),
      pltpu.VMEM((HIST_PAD,), jnp.float32),
  ]
  run = functools.partial(
      pl.kernel,
      mesh=mesh,
      compiler_params=pltpu.CompilerParams(needs_layout_passes=False),
      out_type=jax.ShapeDtypeStruct((rows, HIST_PAD), jnp.float32),
      scratch_types=per_row_scratch + per_row_scratch + [
          pltpu.SemaphoreType.DMA,
          pltpu.SemaphoreType.DMA,
          pltpu.SemaphoreType.DMA,
          pltpu.SemaphoreType.DMA,
      ],
  )(_row_kernel)
  out = run(x2)
  return out[:, :NB * NB].reshape(N, C, NB, NB)


# speedup vs baseline: 25.7858x; 1.0792x over previous
import functools

import jax
import jax.numpy as jnp
from jax import lax
from jax.experimental import pallas as pl
from jax.experimental.pallas import tpu as pltpu
from jax.experimental.pallas import tpu_sc as plsc

L = 4096
NB = 65
HIST_PAD = 4240
NLANE = 16
NVREG = L // NLANE
NCORES = 2
NSUB = 16
NWORKERS = NCORES * NSUB
RADIX = 1 << 11

_I32_MIN = -2147483648
_I32_MAX = 2147483647


def _row_kernel(x_hbm, out_hbm,
                xvA, keys0A, pay0A, keys1A, pay1A, hist0A, hist1A, binsA,
                rowhistA,
                xvB, keys0B, pay0B, keys1B, pay1B, hist0B, hist1B, binsB,
                rowhistB,
                semA, semB, osemA, osemB):
  total_rows = x_hbm.shape[0]
  rows_per_worker = total_rows // NWORKERS
  npairs = rows_per_worker // 2
  wid = lax.axis_index("s") * NCORES + lax.axis_index("c")
  iota = lax.iota(jnp.int32, NLANE)
  zeros16i = jnp.zeros((NLANE,), jnp.int32)
  zeros16f = jnp.zeros((NLANE,), jnp.float32)

  binsA[pl.ds(L, NLANE)] = zeros16i
  binsB[pl.ds(L, NLANE)] = zeros16i

  def pair_body(r, _):
    rowA = wid * rows_per_worker + 2 * r
    rowB = rowA + 1

    @pl.when(r > 0)
    def _():
      pltpu.make_async_copy(rowhistA, out_hbm.at[rowA - 2], osemA).wait()
      pltpu.make_async_copy(rowhistB, out_hbm.at[rowB - 2], osemB).wait()

    cpA = pltpu.make_async_copy(x_hbm.at[rowA], xvA, semA)
    cpB = pltpu.make_async_copy(x_hbm.at[rowB], xvB, semB)
    cpA.start()
    cpB.start()
    cpA.wait()
    cpB.wait()

    def h0clear_body(j, _):
      hist0A[pl.ds(j * NLANE, NLANE)] = zeros16i
      hist0B[pl.ds(j * NLANE, NLANE)] = zeros16i
      return 0

    lax.fori_loop(0, RADIX // NLANE, h0clear_body, 0, unroll=4)

    def keyfast_body(j, carry):
      zA, zB = carry
      idxv = j * NLANE + iota
      vA = xvA[pl.ds(j * NLANE, NLANE)]
      vB = xvB[pl.ds(j * NLANE, NLANE)]
      tA = plsc.bitcast(vA, jnp.int32)
      tB = plsc.bitcast(vB, jnp.int32)
      uA = (tA ^ (lax.shift_right_arithmetic(tA, 31) & _I32_MAX)) ^ _I32_MIN
      uB = (tB ^ (lax.shift_right_arithmetic(tB, 31) & _I32_MAX)) ^ _I32_MIN
      dA = uA & (RADIX - 1)
      dB = uB & (RADIX - 1)
      occA, lastA = plsc.scan_count(dA)
      occB, lastB = plsc.scan_count(dB)
      keys0A[pl.ds(j * NLANE, NLANE)] = uA
      keys0B[pl.ds(j * NLANE, NLANE)] = uB
      pay0A[pl.ds(j * NLANE, NLANE)] = idxv
      pay0B[pl.ds(j * NLANE, NLANE)] = idxv
      plsc.addupdate_scatter(hist0A, [dA], occA, mask=lastA)
      plsc.addupdate_scatter(hist0B, [dB], occB, mask=lastB)
      return zA | (tA + tA == 0), zB | (tB + tB == 0)

    zA, zB = lax.fori_loop(0, NVREG, keyfast_body, (iota < 0, iota < 0),
                           unroll=4)

    def make_slow_path(xv, keys0, hist0):
      def slow_path():
        def valid_body(j, carry):
          fv, lv = carry
          v = xv[pl.ds(j * NLANE, NLANE)]
          nz = v != 0.0
          idxv = j * NLANE + iota
          fv = jnp.minimum(fv, jnp.where(nz, idxv, jnp.int32(L)))
          lv = jnp.maximum(lv, jnp.where(nz, idxv, jnp.int32(-1)))
          return fv, lv

        fv, lv = lax.fori_loop(0, NVREG, valid_body,
                               (zeros16i + L, zeros16i - 1), unroll=4)
        s_, e_ = jnp.min(fv), jnp.max(lv)

        def hclear(j, _):
          hist0[pl.ds(j * NLANE, NLANE)] = zeros16i
          return 0

        lax.fori_loop(0, RADIX // NLANE, hclear, 0, unroll=4)

        def keymask_body(j, _):
          u = keys0[pl.ds(j * NLANE, NLANE)]
          idxv = j * NLANE + iota
          ok = (idxv >= s_) & (idxv <= e_)
          key = jnp.where(ok, u, jnp.int32(-1))
          keys0[pl.ds(j * NLANE, NLANE)] = key
          d = key & (RADIX - 1)
          occ, last = plsc.scan_count(d)
          plsc.addupdate_scatter(hist0, [d], occ, mask=last)
          return 0

        lax.fori_loop(0, NVREG, keymask_body, 0, unroll=4)
        return s_, e_

      return slow_path

    full = lambda: (jnp.int32(0), jnp.int32(L - 1))
    anyzeroA = jnp.max(zA.astype(jnp.int32)) > 0
    anyzeroB = jnp.max(zB.astype(jnp.int32)) > 0
    startA, endA = lax.cond(anyzeroA, make_slow_path(xvA, keys0A, hist0A),
                            full)
    startB, endB = lax.cond(anyzeroB, make_slow_path(xvB, keys0B, hist0B),
                            full)
    vlenA = endA - startA + 1
    vlenB = endB - startB + 1
    lencA = jnp.maximum(vlenA, 1)
    lencB = jnp.maximum(vlenB, 1)

    def exclusive_scan2(srcA, srcB, clrA, clrB, n, m):
      fifteen = zeros16i + (NLANE - 1)

      def body(j, carry):
        cA, cB = carry
        vA = srcA[pl.ds(j * NLANE, NLANE)]
        vB = srcB[pl.ds(j * NLANE, NLANE)]
        incA = plsc.cumsum(vA)
        incB = plsc.cumsum(vB)
        srcA[pl.ds(j * NLANE, NLANE)] = incA - vA + cA
        srcB[pl.ds(j * NLANE, NLANE)] = incB - vB + cB

        @pl.when(j < m // NLANE)
        def _():
          clrA[pl.ds(j * NLANE, NLANE)] = zeros16i
          clrB[pl.ds(j * NLANE, NLANE)] = zeros16i

        totA = jnp.take_along_axis(incA, fifteen, axis=0)
        totB = jnp.take_along_axis(incB, fifteen, axis=0)
        return cA + totA, cB + totB

      lax.fori_loop(0, n // NLANE, body, (zeros16i, zeros16i), unroll=2)

    exclusive_scan2(hist0A, hist0B, hist1A, hist1B, RADIX, RADIX)

    def permute01_pair(j, sh, nbits2, Ar, Br):
      kinA, pinA, koutA, poutA, histA, histnextA = Ar
      kinB, pinB, koutB, poutB, histB, histnextB = Br
      sl = pl.ds(j * NLANE, NLANE)
      kA = kinA[sl]
      kB = kinB[sl]
      pA = pinA[sl]
      pB = pinB[sl]
      dA = lax.shift_right_logical(kA, sh) & (RADIX - 1)
      dB = lax.shift_right_logical(kB, sh) & (RADIX - 1)
      dnA = lax.shift_right_logical(kA, sh + 11) & ((1 << nbits2) - 1)
      dnB = lax.shift_right_logical(kB, sh + 11) & ((1 << nbits2) - 1)
      occA, lastA = plsc.scan_count(dA)
      occB, lastB = plsc.scan_count(dB)
      occnA, lastnA = plsc.scan_count(dnA)
      occnB, lastnB = plsc.scan_count(dnB)
      baseA = plsc.load_gather(histA, [dA])
      baseB = plsc.load_gather(histB, [dB])
      slotA = baseA + occA - 1
      slotB = baseB + occB - 1
      plsc.store_scatter(koutA, [slotA], kA)
      plsc.store_scatter(koutB, [slotB], kB)
      plsc.store_scatter(poutA, [slotA], pA)
      plsc.store_scatter(poutB, [slotB], pB)
      plsc.addupdate_scatter(histA, [dA], occA, mask=lastA)
      plsc.addupdate_scatter(histB, [dB], occB, mask=lastB)
      plsc.addupdate_scatter(histnextA, [dnA], occnA, mask=lastnA)
      plsc.addupdate_scatter(histnextB, [dnB], occnB, mask=lastnB)
      return 0

    def permute0_body(j, _):
      return permute01_pair(
          j, 0, 11,
          (keys0A, pay0A, keys1A, pay1A, hist0A, hist1A),
          (keys0B, pay0B, keys1B, pay1B, hist0B, hist1B))

    lax.fori_loop(0, NVREG, permute0_body, 0, unroll=2)

    exclusive_scan2(hist1A, hist1B, hist0A, hist0B, RADIX, 1024)

    def permute1_body(j, _):
      return permute01_pair(
          j, 11, 10,
          (keys1A, pay1A, keys0A, pay0A, hist1A, hist0A),
          (keys1B, pay1B, keys0B, pay0B, hist1B, hist0B))

    lax.fori_loop(0, NVREG, permute1_body, 0, unroll=2)

    exclusive_scan2(hist0A, hist0B, hist1A, hist1B, 1024, 0)

    def hclear_body(j, _):
      rowhistA[pl.ds(j * NLANE, NLANE)] = zeros16f
      rowhistB[pl.ds(j * NLANE, NLANE)] = zeros16f
      return 0

    lax.fori_loop(0, HIST_PAD // NLANE, hclear_body, 0, unroll=4)

    invlenA = (zeros16f + 1.0) / (zeros16i + lencA).astype(jnp.float32)
    invlenB = (zeros16f + 1.0) / (zeros16i + lencB).astype(jnp.float32)

    def permute2_body(j, _):
      sl = pl.ds(j * NLANE, NLANE)
      kA = keys0A[sl]
      kB = keys0B[sl]
      pA = pay0A[sl]
      pB = pay0B[sl]
      dA = lax.shift_right_logical(kA, 22) & 1023
      dB = lax.shift_right_logical(kB, 22) & 1023
      occA, lastA = plsc.scan_count(dA)
      occB, lastB = plsc.scan_count(dB)
      baseA = plsc.load_gather(hist0A, [dA])
      baseB = plsc.load_gather(hist0B, [dB])
      slotA = baseA + occA - 1
      slotB = baseB + occB - 1
      plsc.addupdate_scatter(hist0A, [dA], occA, mask=lastA)
      plsc.addupdate_scatter(hist0B, [dB], occB, mask=lastB)
      bfA = (slotA * NB).astype(jnp.float32) * invlenA + 5e-5
      bfB = (slotB * NB).astype(jnp.float32) * invlenB + 5e-5
      bA = jnp.minimum(bfA.astype(jnp.int32), NB - 1)
      bB = jnp.minimum(bfB.astype(jnp.int32), NB - 1)
      plsc.store_scatter(binsA, [pA], bA)
      plsc.store_scatter(binsB, [pB], bB)
      return 0

    lax.fori_loop(0, NVREG, permute2_body, 0, unroll=2)

    invA = (zeros16f + 1.0) / \
        (zeros16i + jnp.maximum(vlenA - 1, 1)).astype(jnp.float32)
    invB = (zeros16f + 1.0) / \
        (zeros16i + jnp.maximum(vlenB - 1, 1)).astype(jnp.float32)

    def trans_masked_one(j, bins, rowhist, start, end, inv):
      a = bins[pl.ds(j * NLANE, NLANE)]
      b = bins[pl.ds(j * NLANE + 1, NLANE)]
      t = j * NLANE + iota
      ok = (t >= start) & (t <= end - 1)
      cell = a * NB + b
      occ, last = plsc.scan_count(cell, mask=ok)
      plsc.addupdate_scatter(rowhist, [cell], occ.astype(jnp.float32) * inv,
                             mask=last & ok)
      return 0

    def trans_all_masked():
      def body(j, _):
        trans_masked_one(j, binsA, rowhistA, startA, endA, invA)
        trans_masked_one(j, binsB, rowhistB, startB, endB, invB)
        return 0

      lax.fori_loop(0, NVREG, body, 0, unroll=2)
      return 0

    def trans_all_fast():
      def body(j, _):
        sl = pl.ds(j * NLANE, NLANE)
        sl1 = pl.ds(j * NLANE + 1, NLANE)
        aA = binsA[sl]
        aB = binsB[sl]
        bA = binsA[sl1]
        bB = binsB[sl1]
        cellA = aA * NB + bA
        cellB = aB * NB + bB
        occA, lastA = plsc.scan_count(cellA)
        occB, lastB = plsc.scan_count(cellB)
        plsc.addupdate_scatter(rowhistA, [cellA],
                               occA.astype(jnp.float32) * invA, mask=lastA)
        plsc.addupdate_scatter(rowhistB, [cellB],
                               occB.astype(jnp.float32) * invB, mask=lastB)
        return 0

      lax.fori_loop(0, NVREG - 1, body, 0, unroll=2)
      trans_masked_one(NVREG - 1, binsA, rowhistA, startA, endA, invA)
      trans_masked_one(NVREG - 1, binsB, rowhistB, startB, endB, invB)
      return 0

    lax.cond(anyzeroA | anyzeroB, trans_all_masked, trans_all_fast)

    pltpu.make_async_copy(rowhistA, out_hbm.at[rowA], osemA).start()
    pltpu.make_async_copy(rowhistB, out_hbm.at[rowB], osemB).start()
    return 0

  lax.fori_loop(0, npairs, pair_body, 0)
  last_rowA = wid * rows_per_worker + 2 * (npairs - 1)
  pltpu.make_async_copy(rowhistA, out_hbm.at[last_rowA], osemA).wait()
  pltpu.make_async_copy(rowhistB, out_hbm.at[last_rowA + 1], osemB).wait()


@jax.jit
def kernel(x):
  N, C, Lx = x.shape
  rows = N * C
  x2 = x.reshape(rows, Lx)
  mesh = plsc.VectorSubcoreMesh(core_axis_name="c", subcore_axis_name="s",
                                num_cores=NCORES, num_subcores=NSUB)
  per_row_scratch = [
      pltpu.VMEM((L,), jnp.float32),
      pltpu.VMEM((L,), jnp.int32),
      pltpu.VMEM((L,), jnp.int32),
      pltpu.VMEM((L,), jnp.int32),
      pltpu.VMEM((L,), jnp.int32),
      pltpu.VMEM((RADIX,), jnp.int32),
      pltpu.VMEM((RADIX,), jnp.int32),
      pltpu.VMEM((L + NLANE,), jnp.int32),
      pltpu.VMEM((HIST_PAD,), jnp.float32),
  ]
  run = functools.partial(
      pl.kernel,
      mesh=mesh,
      compiler_params=pltpu.CompilerParams(needs_layout_passes=False),
      out_type=jax.ShapeDtypeStruct((rows, HIST_PAD), jnp.float32),
      scratch_types=per_row_scratch + per_row_scratch + [
          pltpu.SemaphoreType.DMA,
          pltpu.SemaphoreType.DMA,
          pltpu.SemaphoreType.DMA,
          pltpu.SemaphoreType.DMA,
      ],
  )(_row_kernel)
  out = run(x2)
  return out[:, :NB * NB].reshape(N, C, NB, NB)
